# baseline (speedup 1.0000x reference)
import functools

import jax
import jax.numpy as jnp
from jax import lax
from jax.experimental import pallas as pl
from jax.experimental.pallas import tpu as pltpu
from jax.experimental.pallas import tpu_sc as plsc

N_PIX = 76800
NPT = 9
NCR = 8
NROW = NPT * NCR
VOL = 256
NW = 32
QW = N_PIX // NW
SEG = 48
NSEG = QW // SEG
FPR = 64
PB_A = 1024
PB_B = 2400


def _fdiv(x, c):
    return jnp.floor((x + 0.5) * (1.0 / c))


def _corner_bits(corner):
    i = _fdiv(corner, 4.0)
    rem = corner - 4.0 * i
    j = _fdiv(rem, 2.0)
    k = rem - 2.0 * j
    return i, j, k


def _expand_a_kernel(cx, cy, cz, dx, dy, dz,
                     t216r, bit216r, m0216r, m1216r,
                     t72r, b720r, b721r, b722r,
                     bit24r, m024r, m124r, b80r, b81r, b82r,
                     ind_ref, w_ref, inde_ref, we_ref):
    c = [cx[...], cy[...], cz[...]]
    d = [dx[...], dy[...], dz[...]]

    m0 = m0216r[...] > 0.5
    m1 = m1216r[...] > 0.5
    csel = jnp.where(m0, c[0], jnp.where(m1, c[1], c[2]))
    dsel = jnp.where(m0, d[0], jnp.where(m1, d[1], d[2]))
    pt = csel + t216r[...] * dsel
    fl = jnp.floor(pt)
    nb = jnp.sign(pt - fl)
    ind_ref[...] = (fl + bit216r[...] * nb).astype(jnp.int32)

    b72 = [b720r[...] > 0.5, b721r[...] > 0.5, b722r[...] > 0.5]
    w = None
    for a in range(3):
        pta = c[a] + t72r[...] * d[a]
        fla = jnp.floor(pta)
        ala = jnp.abs(pta - fla)
        fac = jnp.where(b72[a], ala, 1.0 - ala)
        w = fac if w is None else w * fac
    w_ref[...] = w

    fle = [jnp.floor(c[a] - 5.0 * d[a]) for a in range(3)]
    nbe = [jnp.sign((c[a] - 5.0 * d[a]) - fle[a]) for a in range(3)]
    ale = [jnp.abs((c[a] - 5.0 * d[a]) - fle[a]) for a in range(3)]

    me0 = m024r[...] > 0.5
    me1 = m124r[...] > 0.5
    flsel = jnp.where(me0, fle[0], jnp.where(me1, fle[1], fle[2]))
    nbsel = jnp.where(me0, nbe[0], jnp.where(me1, nbe[1], nbe[2]))
    inde_ref[...] = (flsel + bit24r[...] * nbsel).astype(jnp.int32)

    b8 = [b80r[...] > 0.5, b81r[...] > 0.5, b82r[...] > 0.5]
    we = None
    for a in range(3):
        fac = jnp.where(b8[a], ale[a], 1.0 - ale[a])
        we = fac if we is None else we * fac
    we_ref[...] = we


def _expand_b_kernel(cx, cy, cz, dx, dy, dz, lin_ref, wm_ref, flags_ref):
    c = [cx[0, 0], cy[0, 0], cz[0, 0]]
    d = [dx[0, 0], dy[0, 0], dz[0, 0]]

    r72i = lax.broadcasted_iota(jnp.int32, (NROW, PB_B), 0)
    r72 = r72i.astype(jnp.float32)
    corner = _fdiv(r72, 9.0)
    t = (r72 - 9.0 * corner) - 4.0
    bits = _corner_bits(corner)
    pid = pl.program_id(0)
    qglob = ((pid // (QW // PB_B)) * QW + (pid % (QW // PB_B)) * PB_B
             + lax.broadcasted_iota(jnp.int32, (NROW, PB_B), 1))
    spread = ((r72i * N_PIX + qglob) * 16) & (VOL * VOL * VOL - 1)

    lin = None
    wprod = None
    valid = None
    for a in range(3):
        pta = c[a] + t * d[a]
        fla = jnp.floor(pta)
        dfa = pta - fla
        iva = fla + bits[a] * jnp.sign(dfa)
        ia = iva.astype(jnp.int32)
        va = (ia >= 0) & (ia < VOL)
        cla = jnp.clip(ia, 0, VOL - 1)
        ala = jnp.abs(dfa)
        fac = jnp.where(bits[a] > 0.5, ala, 1.0 - ala)
        wprod = fac if wprod is None else wprod * fac
        valid = va if valid is None else valid & va
        lin = cla if lin is None else lin * VOL + cla
    wm = jnp.where(valid, wprod, 0.0)
    lin_ref[0] = jnp.where(valid, lin, spread)
    wm_ref[0] = wm
    nz = (wm != 0.0).astype(jnp.float32)
    cols = [jnp.max(nz[:, k * SEG:(k + 1) * SEG], axis=1) for k in range(NSEG)]
    cols += [jnp.zeros((NROW,), jnp.float32)] * (FPR - NSEG)
    flags_ref[0] = (jnp.stack(cols, axis=1) > 0.5).astype(jnp.int32)


def _tables():
    import numpy as np
    k = np.arange(216)
    point, rem = k // 24, k % 24
    corner, axis = rem // 3, rem % 3
    t216 = (point - 4).astype(np.float32)
    bit216 = ((corner >> (2 - axis)) & 1).astype(np.float32)
    m0216 = (axis == 0).astype(np.float32)
    m1216 = (axis == 1).astype(np.float32)
    k = np.arange(72)
    t72 = (k // 8 - 4).astype(np.float32)
    c72 = k % 8
    b72 = [((c72 >> (2 - a)) & 1).astype(np.float32) for a in range(3)]
    k = np.arange(24)
    c24, a24 = k // 3, k % 3
    bit24 = ((c24 >> (2 - a24)) & 1).astype(np.float32)
    m024 = (a24 == 0).astype(np.float32)
    m124 = (a24 == 1).astype(np.float32)
    c8 = np.arange(8)
    b8 = [((c8 >> (2 - a)) & 1).astype(np.float32) for a in range(3)]
    rows = [t216, bit216, m0216, m1216, t72] + b72 + [bit24, m024, m124] + b8
    return [jnp.asarray(r.reshape(-1, 1)) for r in rows]


def _expand_a(cB, dB):
    tabs = _tables()
    ins = list(cB) + list(dB) + tabs
    in_spec = pl.BlockSpec((PB_A,), lambda i: (i,))
    tab_specs = [pl.BlockSpec((t.shape[0], 1), lambda i: (0, 0))
                 for t in tabs]
    grid = (N_PIX // PB_A,)
    return pl.pallas_call(
        _expand_a_kernel,
        grid=grid,
        in_specs=[in_spec] * 6 + tab_specs,
        out_specs=[
            pl.BlockSpec((216, PB_A), lambda i: (0, i)),
            pl.BlockSpec((72, PB_A), lambda i: (0, i)),
            pl.BlockSpec((24, PB_A), lambda i: (0, i)),
            pl.BlockSpec((8, PB_A), lambda i: (0, i)),
        ],
        out_shape=[
            jax.ShapeDtypeStruct((216, N_PIX), jnp.int32),
            jax.ShapeDtypeStruct((72, N_PIX), jnp.float32),
            jax.ShapeDtypeStruct((24, N_PIX), jnp.int32),
            jax.ShapeDtypeStruct((8, N_PIX), jnp.float32),
        ],
    )(*ins)


def _expand_b(cB, dB):
    ins = [x.reshape(NW, 1, PB_B) for x in list(cB) + list(dB)]
    in_spec = pl.BlockSpec((1, 1, PB_B), lambda i: (i, 0, 0))
    nsub = QW // PB_B
    grid = (N_PIX // PB_B,)
    out_spec = pl.BlockSpec((1, NROW, PB_B), lambda i: (i, 0, 0))
    flag_spec = pl.BlockSpec((1, NROW, FPR), lambda i: (i, 0, 0))
    return pl.pallas_call(
        _expand_b_kernel,
        grid=grid,
        in_specs=[in_spec] * 6,
        out_specs=[out_spec, out_spec, flag_spec],
        out_shape=[
            jax.ShapeDtypeStruct((NW, NROW, QW), jnp.int32),
            jax.ShapeDtypeStruct((NW, NROW, QW), jnp.float32),
            jax.ShapeDtypeStruct((NW, NROW, FPR), jnp.int32),
        ],
    )(*ins)


GRP = 1
NGRP = NROW // GRP


def _sc_fusion_kernel(lin_hbm, wm_hbm, tsdf_hbm, wv_hbm, flags_hbm,
                      fv_hbm, fw_hbm,
                      idx0, idx1, wm0, wm1, gvt, gvw, flagb, sfv, sfw,
                      sem0, sem1, semg):
    cid = lax.axis_index("c")
    sid = lax.axis_index("s")
    wid = sid * 2 + cid
    q0 = wid * QW
    wbase = wid * (NROW * QW)

    def zbody(j, _):
        z = jnp.zeros((16,), jnp.float32)
        sfv[pl.ds(j * 16, 16)] = z
        sfw[pl.ds(j * 16, 16)] = z
        return 0

    lax.fori_loop(0, QW * NPT // 16, zbody, 0)

    pltpu.sync_copy(flags_hbm.at[pl.ds(wid * NROW * FPR, NROW * FPR)], flagb)

    def fire_grp(g, idx_v, wm_v, sem):
        off = wbase + g * (GRP * QW)
        pltpu.async_copy(lin_hbm.at[pl.ds(off, GRP * QW)], idx_v, sem)
        pltpu.async_copy(wm_hbm.at[pl.ds(off, GRP * QW)], wm_v, sem)

    def proc_grp(g, idx_v, wm_v, sem):
        pltpu.make_async_copy(lin_hbm.at[pl.ds(0, GRP * QW)], idx_v,
                              sem).wait()
        pltpu.make_async_copy(wm_hbm.at[pl.ds(0, GRP * QW)], wm_v,
                              sem).wait()
        for j in range(GRP):
            r = g * GRP + j
            tt = lax.rem(r, NPT)
            flv = [flagb[pl.ds(r * FPR + 16 * v, 16)] for v in range(4)]

            def flag(k):
                return flv[k // 16][k % 16] != 0

            for k in range(NSEG):
                @pl.when(flag(k))
                def _(k=k):
                    isl = pl.ds(j * QW + k * SEG, SEG)
                    ssl = pl.ds(k * SEG, SEG)
                    pltpu.async_copy(tsdf_hbm.at[idx_v.at[isl]], gvt.at[ssl],
                                     semg)
                    pltpu.async_copy(wv_hbm.at[idx_v.at[isl]], gvw.at[ssl],
                                     semg)

            for k in range(NSEG):
                @pl.when(flag(k))
                def _(k=k):
                    ssl = pl.ds(k * SEG, SEG)
                    pltpu.make_async_copy(tsdf_hbm.at[pl.ds(0, SEG)],
                                          gvt.at[ssl], semg).wait()
                    pltpu.make_async_copy(wv_hbm.at[pl.ds(0, SEG)],
                                          gvw.at[ssl], semg).wait()

            for k in range(NSEG):
                @pl.when(flag(k))
                def _(k=k):
                    def ub(u, _2):
                        sl = pl.ds(j * QW + k * SEG + u * 16, 16)
                        gsl = pl.ds(k * SEG + u * 16, 16)
                        osl = pl.ds(tt * QW + k * SEG + u * 16, 16)
                        wmv = wm_v[sl]
                        sfv[osl] = sfv[osl] + wmv * gvt[gsl]
                        sfw[osl] = sfw[osl] + wmv * gvw[gsl]
                        return 0

                    lax.fori_loop(0, SEG // 16, ub, 0)

    fire_grp(0, idx0, wm0, sem0)

    def ibody(i, _):
        fire_grp(2 * i + 1, idx1, wm1, sem1)
        proc_grp(2 * i, idx0, wm0, sem0)

        @pl.when(i < NGRP // 2 - 1)
        def _():
            fire_grp(2 * i + 2, idx0, wm0, sem0)

        proc_grp(2 * i + 1, idx1, wm1, sem1)
        return 0

    lax.fori_loop(0, NGRP // 2, ibody, 0)

    def obody(tt, _):
        pltpu.sync_copy(sfv.at[pl.ds(tt * QW, QW)],
                        fv_hbm.at[pl.ds(tt * N_PIX + q0, QW)])
        pltpu.sync_copy(sfw.at[pl.ds(tt * QW, QW)],
                        fw_hbm.at[pl.ds(tt * N_PIX + q0, QW)])
        return 0

    lax.fori_loop(0, NPT, obody, 0)


def _fusion(lin_flat, wm_flat, tsdf_flat, wv_flat, flags_flat):
    mesh = plsc.VectorSubcoreMesh(core_axis_name="c", subcore_axis_name="s")
    f = functools.partial(
        pl.kernel,
        mesh=mesh,
        out_type=[
            jax.ShapeDtypeStruct((NPT * N_PIX,), jnp.float32),
            jax.ShapeDtypeStruct((NPT * N_PIX,), jnp.float32),
        ],
        scratch_types=[
            pltpu.VMEM((GRP * QW,), jnp.int32),
            pltpu.VMEM((GRP * QW,), jnp.int32),
            pltpu.VMEM((GRP * QW,), jnp.float32),
            pltpu.VMEM((GRP * QW,), jnp.float32),
            pltpu.VMEM((QW,), jnp.float32),
            pltpu.VMEM((QW,), jnp.float32),
            pltpu.VMEM((NROW * FPR,), jnp.int32),
            pltpu.VMEM((QW * NPT,), jnp.float32),
            pltpu.VMEM((QW * NPT,), jnp.float32),
            pltpu.SemaphoreType.DMA,
            pltpu.SemaphoreType.DMA,
            pltpu.SemaphoreType.DMA,
        ],
    )(_sc_fusion_kernel)
    return f(lin_flat, wm_flat, tsdf_flat, wv_flat, flags_flat)


def kernel(depth, extrinsics, intrinsics, tsdf_volume, feature_volume,
           origin, resolution, gpu, weights_volume):
    intr = intrinsics.astype(jnp.float32)
    extr = extrinsics.astype(jnp.float32)

    b, h, w = depth.shape
    n = h * w
    xx, yy = jnp.meshgrid(jnp.arange(h, dtype=jnp.float32),
                          jnp.arange(w, dtype=jnp.float32), indexing='ij')
    xx = jnp.tile(xx.reshape(1, n, 1), (b, 1, 1))
    yy = jnp.tile(yy.reshape(1, n, 1), (b, 1, 1))
    zz = depth.reshape(b, n, 1)
    points_p = jnp.concatenate([yy * zz, xx * zz, zz], axis=2)
    intrinsics_inv = jnp.linalg.inv(intr)
    points_c = jnp.matmul(intrinsics_inv, jnp.transpose(points_p, (0, 2, 1)))
    hom = jnp.ones((b, 1, n), dtype=jnp.float32)
    points_c = jnp.concatenate([points_c, hom], axis=1)
    points_w = jnp.matmul(extr[:3], points_c)

    eye_w = extr[:, :3, 3]
    eye_v = (eye_w - origin) / resolution
    cw = [points_w[0, a] for a in range(3)]
    cB = [(cw[a] - origin[a]) / resolution for a in range(3)]
    dirc = [cB[a] - eye_v[0, a] for a in range(3)]
    norm = jnp.sqrt((dirc[0] * dirc[0] + dirc[1] * dirc[1])
                    + dirc[2] * dirc[2])
    nrm = jnp.maximum(norm, 1e-12)
    dB = [dirc[a] / nrm for a in range(3)]

    ind216, w72, inde24, we8 = _expand_a(cB, dB)
    lin_wb, wm_wb, flags = _expand_b(cB, dB)

    fv_tm, fw_tm = _fusion(
        lin_wb.reshape(-1), wm_wb.reshape(-1),
        tsdf_volume.reshape(-1), weights_volume.reshape(-1),
        flags.reshape(-1))
    fv = fv_tm.reshape(NPT, N_PIX).T
    fw = fw_tm.reshape(NPT, N_PIX).T

    return (fv.reshape(1, N_PIX, NPT),
            fw.reshape(1, N_PIX, NPT),
            ind216.T.reshape(1, N_PIX, NPT, NCR, 3),
            w72.T.reshape(1, N_PIX, NPT, NCR),
            inde24.T.reshape(1, N_PIX, 1, NCR, 3),
            we8.T.reshape(1, N_PIX, 1, NCR))


# speedup vs baseline: 1.2271x; 1.2271x over previous
import functools

import jax
import jax.numpy as jnp
from jax import lax
from jax.experimental import pallas as pl
from jax.experimental.pallas import tpu as pltpu
from jax.experimental.pallas import tpu_sc as plsc

N_PIX = 76800
NPT = 9
NCR = 8
NROW = NPT * NCR
VOL = 256
NW = 32
QW = N_PIX // NW
SEG = 240
NSEG = QW // SEG
FPR = 16
PB_A = 1024
PB_B = 2400


def _fdiv(x, c):
    return jnp.floor((x + 0.5) * (1.0 / c))


def _corner_bits(corner):
    i = _fdiv(corner, 4.0)
    rem = corner - 4.0 * i
    j = _fdiv(rem, 2.0)
    k = rem - 2.0 * j
    return i, j, k


def _expand_a_kernel(cx, cy, cz, dx, dy, dz,
                     t216r, bit216r, m0216r, m1216r,
                     t72r, b720r, b721r, b722r,
                     bit24r, m024r, m124r, b80r, b81r, b82r,
                     ind_ref, w_ref, inde_ref, we_ref):
    c = [cx[...], cy[...], cz[...]]
    d = [dx[...], dy[...], dz[...]]

    m0 = m0216r[...] > 0.5
    m1 = m1216r[...] > 0.5
    csel = jnp.where(m0, c[0], jnp.where(m1, c[1], c[2]))
    dsel = jnp.where(m0, d[0], jnp.where(m1, d[1], d[2]))
    pt = csel + t216r[...] * dsel
    fl = jnp.floor(pt)
    nb = jnp.sign(pt - fl)
    ind_ref[...] = (fl + bit216r[...] * nb).astype(jnp.int32)

    b72 = [b720r[...] > 0.5, b721r[...] > 0.5, b722r[...] > 0.5]
    w = None
    for a in range(3):
        pta = c[a] + t72r[...] * d[a]
        fla = jnp.floor(pta)
        ala = jnp.abs(pta - fla)
        fac = jnp.where(b72[a], ala, 1.0 - ala)
        w = fac if w is None else w * fac
    w_ref[...] = w

    fle = [jnp.floor(c[a] - 5.0 * d[a]) for a in range(3)]
    nbe = [jnp.sign((c[a] - 5.0 * d[a]) - fle[a]) for a in range(3)]
    ale = [jnp.abs((c[a] - 5.0 * d[a]) - fle[a]) for a in range(3)]

    me0 = m024r[...] > 0.5
    me1 = m124r[...] > 0.5
    flsel = jnp.where(me0, fle[0], jnp.where(me1, fle[1], fle[2]))
    nbsel = jnp.where(me0, nbe[0], jnp.where(me1, nbe[1], nbe[2]))
    inde_ref[...] = (flsel + bit24r[...] * nbsel).astype(jnp.int32)

    b8 = [b80r[...] > 0.5, b81r[...] > 0.5, b82r[...] > 0.5]
    we = None
    for a in range(3):
        fac = jnp.where(b8[a], ale[a], 1.0 - ale[a])
        we = fac if we is None else we * fac
    we_ref[...] = we


def _expand_b_kernel(cx, cy, cz, dx, dy, dz, lin_ref, wm_ref, flags_ref):
    c = [cx[0, 0], cy[0, 0], cz[0, 0]]
    d = [dx[0, 0], dy[0, 0], dz[0, 0]]

    r72i = lax.broadcasted_iota(jnp.int32, (NROW, PB_B), 0)
    r72 = r72i.astype(jnp.float32)
    corner = _fdiv(r72, 9.0)
    t = (r72 - 9.0 * corner) - 4.0
    bits = _corner_bits(corner)
    pid = pl.program_id(0)
    qglob = ((pid // (QW // PB_B)) * QW + (pid % (QW // PB_B)) * PB_B
             + lax.broadcasted_iota(jnp.int32, (NROW, PB_B), 1))
    spread = ((r72i * N_PIX + qglob) * 16) & (VOL * VOL * VOL - 1)

    lin = None
    wprod = None
    valid = None
    for a in range(3):
        pta = c[a] + t * d[a]
        fla = jnp.floor(pta)
        dfa = pta - fla
        iva = fla + bits[a] * jnp.sign(dfa)
        ia = iva.astype(jnp.int32)
        va = (ia >= 0) & (ia < VOL)
        cla = jnp.clip(ia, 0, VOL - 1)
        ala = jnp.abs(dfa)
        fac = jnp.where(bits[a] > 0.5, ala, 1.0 - ala)
        wprod = fac if wprod is None else wprod * fac
        valid = va if valid is None else valid & va
        lin = cla if lin is None else lin * VOL + cla
    wm = jnp.where(valid, wprod, 0.0)
    lin_ref[0] = jnp.where(valid, lin, spread)
    wm_ref[0] = wm
    nz = (wm != 0.0).astype(jnp.float32)
    cols = [jnp.max(nz[:, k * SEG:(k + 1) * SEG], axis=1) for k in range(NSEG)]
    cols += [jnp.zeros((NROW,), jnp.float32)] * (FPR - NSEG)
    flags_ref[0] = (jnp.stack(cols, axis=1) > 0.5).astype(jnp.int32)


def _tables():
    import numpy as np
    k = np.arange(216)
    point, rem = k // 24, k % 24
    corner, axis = rem // 3, rem % 3
    t216 = (point - 4).astype(np.float32)
    bit216 = ((corner >> (2 - axis)) & 1).astype(np.float32)
    m0216 = (axis == 0).astype(np.float32)
    m1216 = (axis == 1).astype(np.float32)
    k = np.arange(72)
    t72 = (k // 8 - 4).astype(np.float32)
    c72 = k % 8
    b72 = [((c72 >> (2 - a)) & 1).astype(np.float32) for a in range(3)]
    k = np.arange(24)
    c24, a24 = k // 3, k % 3
    bit24 = ((c24 >> (2 - a24)) & 1).astype(np.float32)
    m024 = (a24 == 0).astype(np.float32)
    m124 = (a24 == 1).astype(np.float32)
    c8 = np.arange(8)
    b8 = [((c8 >> (2 - a)) & 1).astype(np.float32) for a in range(3)]
    rows = [t216, bit216, m0216, m1216, t72] + b72 + [bit24, m024, m124] + b8
    return [jnp.asarray(r.reshape(-1, 1)) for r in rows]


def _expand_a(cB, dB):
    tabs = _tables()
    ins = list(cB) + list(dB) + tabs
    in_spec = pl.BlockSpec((PB_A,), lambda i: (i,))
    tab_specs = [pl.BlockSpec((t.shape[0], 1), lambda i: (0, 0))
                 for t in tabs]
    grid = (N_PIX // PB_A,)
    return pl.pallas_call(
        _expand_a_kernel,
        grid=grid,
        in_specs=[in_spec] * 6 + tab_specs,
        out_specs=[
            pl.BlockSpec((216, PB_A), lambda i: (0, i)),
            pl.BlockSpec((72, PB_A), lambda i: (0, i)),
            pl.BlockSpec((24, PB_A), lambda i: (0, i)),
            pl.BlockSpec((8, PB_A), lambda i: (0, i)),
        ],
        out_shape=[
            jax.ShapeDtypeStruct((216, N_PIX), jnp.int32),
            jax.ShapeDtypeStruct((72, N_PIX), jnp.float32),
            jax.ShapeDtypeStruct((24, N_PIX), jnp.int32),
            jax.ShapeDtypeStruct((8, N_PIX), jnp.float32),
        ],
    )(*ins)


def _expand_b(cB, dB):
    ins = [x.reshape(NW, 1, PB_B) for x in list(cB) + list(dB)]
    in_spec = pl.BlockSpec((1, 1, PB_B), lambda i: (i, 0, 0))
    nsub = QW // PB_B
    grid = (N_PIX // PB_B,)
    out_spec = pl.BlockSpec((1, NROW, PB_B), lambda i: (i, 0, 0))
    flag_spec = pl.BlockSpec((1, NROW, FPR), lambda i: (i, 0, 0))
    return pl.pallas_call(
        _expand_b_kernel,
        grid=grid,
        in_specs=[in_spec] * 6,
        out_specs=[out_spec, out_spec, flag_spec],
        out_shape=[
            jax.ShapeDtypeStruct((NW, NROW, QW), jnp.int32),
            jax.ShapeDtypeStruct((NW, NROW, QW), jnp.float32),
            jax.ShapeDtypeStruct((NW, NROW, FPR), jnp.int32),
        ],
    )(*ins)


GRP = 4
NGRP = NROW // GRP


def _sc_fusion_kernel(lin_hbm, wm_hbm, tsdf_hbm, wv_hbm, flags_hbm,
                      fv_hbm, fw_hbm,
                      idx0, idx1, wm0, wm1, gvt, gvw, flagb, sfv, sfw,
                      sem0, sem1, semg):
    cid = lax.axis_index("c")
    sid = lax.axis_index("s")
    wid = sid * 2 + cid
    q0 = wid * QW
    wbase = wid * (NROW * QW)

    def zbody(j, _):
        z = jnp.zeros((16,), jnp.float32)
        sfv[pl.ds(j * 16, 16)] = z
        sfw[pl.ds(j * 16, 16)] = z
        return 0

    lax.fori_loop(0, QW * NPT // 16, zbody, 0)

    pltpu.sync_copy(flags_hbm.at[pl.ds(wid * NROW * FPR, NROW * FPR)], flagb)

    def fire_grp(g, idx_v, wm_v, sem):
        off = wbase + g * (GRP * QW)
        pltpu.async_copy(lin_hbm.at[pl.ds(off, GRP * QW)], idx_v, sem)
        pltpu.async_copy(wm_hbm.at[pl.ds(off, GRP * QW)], wm_v, sem)

    def proc_grp(g, idx_v, wm_v, sem):
        pltpu.make_async_copy(lin_hbm.at[pl.ds(0, GRP * QW)], idx_v,
                              sem).wait()
        pltpu.make_async_copy(wm_hbm.at[pl.ds(0, GRP * QW)], wm_v,
                              sem).wait()
        for j in range(GRP):
            r = g * GRP + j
            tt = lax.rem(r, NPT)
            fl16 = flagb[pl.ds(r * FPR, 16)]

            for k in range(NSEG):
                @pl.when(fl16[k] != 0)
                def _(k=k):
                    isl = pl.ds(j * QW + k * SEG, SEG)
                    ssl = pl.ds(k * SEG, SEG)
                    pltpu.async_copy(tsdf_hbm.at[idx_v.at[isl]], gvt.at[ssl],
                                     semg)
                    pltpu.async_copy(wv_hbm.at[idx_v.at[isl]], gvw.at[ssl],
                                     semg)

            for k in range(NSEG):
                @pl.when(fl16[k] != 0)
                def _(k=k):
                    ssl = pl.ds(k * SEG, SEG)
                    pltpu.make_async_copy(tsdf_hbm.at[pl.ds(0, SEG)],
                                          gvt.at[ssl], semg).wait()
                    pltpu.make_async_copy(wv_hbm.at[pl.ds(0, SEG)],
                                          gvw.at[ssl], semg).wait()

            for k in range(NSEG):
                @pl.when(fl16[k] != 0)
                def _(k=k):
                    def ub(u, _2):
                        sl = pl.ds(j * QW + k * SEG + u * 16, 16)
                        gsl = pl.ds(k * SEG + u * 16, 16)
                        osl = pl.ds(tt * QW + k * SEG + u * 16, 16)
                        wmv = wm_v[sl]
                        sfv[osl] = sfv[osl] + wmv * gvt[gsl]
                        sfw[osl] = sfw[osl] + wmv * gvw[gsl]
                        return 0

                    lax.fori_loop(0, SEG // 16, ub, 0)

    fire_grp(0, idx0, wm0, sem0)

    def ibody(i, _):
        fire_grp(2 * i + 1, idx1, wm1, sem1)
        proc_grp(2 * i, idx0, wm0, sem0)

        @pl.when(i < NGRP // 2 - 1)
        def _():
            fire_grp(2 * i + 2, idx0, wm0, sem0)

        proc_grp(2 * i + 1, idx1, wm1, sem1)
        return 0

    lax.fori_loop(0, NGRP // 2, ibody, 0)

    def obody(tt, _):
        pltpu.sync_copy(sfv.at[pl.ds(tt * QW, QW)],
                        fv_hbm.at[pl.ds(tt * N_PIX + q0, QW)])
        pltpu.sync_copy(sfw.at[pl.ds(tt * QW, QW)],
                        fw_hbm.at[pl.ds(tt * N_PIX + q0, QW)])
        return 0

    lax.fori_loop(0, NPT, obody, 0)


def _fusion(lin_flat, wm_flat, tsdf_flat, wv_flat, flags_flat):
    mesh = plsc.VectorSubcoreMesh(core_axis_name="c", subcore_axis_name="s")
    f = functools.partial(
        pl.kernel,
        mesh=mesh,
        out_type=[
            jax.ShapeDtypeStruct((NPT * N_PIX,), jnp.float32),
            jax.ShapeDtypeStruct((NPT * N_PIX,), jnp.float32),
        ],
        scratch_types=[
            pltpu.VMEM((GRP * QW,), jnp.int32),
            pltpu.VMEM((GRP * QW,), jnp.int32),
            pltpu.VMEM((GRP * QW,), jnp.float32),
            pltpu.VMEM((GRP * QW,), jnp.float32),
            pltpu.VMEM((QW,), jnp.float32),
            pltpu.VMEM((QW,), jnp.float32),
            pltpu.VMEM((NROW * FPR,), jnp.int32),
            pltpu.VMEM((QW * NPT,), jnp.float32),
            pltpu.VMEM((QW * NPT,), jnp.float32),
            pltpu.SemaphoreType.DMA,
            pltpu.SemaphoreType.DMA,
            pltpu.SemaphoreType.DMA,
        ],
    )(_sc_fusion_kernel)
    return f(lin_flat, wm_flat, tsdf_flat, wv_flat, flags_flat)


def kernel(depth, extrinsics, intrinsics, tsdf_volume, feature_volume,
           origin, resolution, gpu, weights_volume):
    intr = intrinsics.astype(jnp.float32)
    extr = extrinsics.astype(jnp.float32)

    b, h, w = depth.shape
    n = h * w
    xx, yy = jnp.meshgrid(jnp.arange(h, dtype=jnp.float32),
                          jnp.arange(w, dtype=jnp.float32), indexing='ij')
    xx = jnp.tile(xx.reshape(1, n, 1), (b, 1, 1))
    yy = jnp.tile(yy.reshape(1, n, 1), (b, 1, 1))
    zz = depth.reshape(b, n, 1)
    points_p = jnp.concatenate([yy * zz, xx * zz, zz], axis=2)
    intrinsics_inv = jnp.linalg.inv(intr)
    points_c = jnp.matmul(intrinsics_inv, jnp.transpose(points_p, (0, 2, 1)))
    hom = jnp.ones((b, 1, n), dtype=jnp.float32)
    points_c = jnp.concatenate([points_c, hom], axis=1)
    points_w = jnp.matmul(extr[:3], points_c)

    eye_w = extr[:, :3, 3]
    eye_v = (eye_w - origin) / resolution
    cw = [points_w[0, a] for a in range(3)]
    cB = [(cw[a] - origin[a]) / resolution for a in range(3)]
    dirc = [cB[a] - eye_v[0, a] for a in range(3)]
    norm = jnp.sqrt((dirc[0] * dirc[0] + dirc[1] * dirc[1])
                    + dirc[2] * dirc[2])
    nrm = jnp.maximum(norm, 1e-12)
    dB = [dirc[a] / nrm for a in range(3)]

    ind216, w72, inde24, we8 = _expand_a(cB, dB)
    lin_wb, wm_wb, flags = _expand_b(cB, dB)

    fv_tm, fw_tm = _fusion(
        lin_wb.reshape(-1), wm_wb.reshape(-1),
        tsdf_volume.reshape(-1), weights_volume.reshape(-1),
        flags.reshape(-1))
    fv = fv_tm.reshape(NPT, N_PIX).T
    fw = fw_tm.reshape(NPT, N_PIX).T

    return (fv.reshape(1, N_PIX, NPT),
            fw.reshape(1, N_PIX, NPT),
            ind216.T.reshape(1, N_PIX, NPT, NCR, 3),
            w72.T.reshape(1, N_PIX, NPT, NCR),
            inde24.T.reshape(1, N_PIX, 1, NCR, 3),
            we8.T.reshape(1, N_PIX, 1, NCR))
